# baseline (speedup 1.0000x reference)
import functools

import jax
import jax.numpy as jnp
from jax import lax
from jax.experimental import pallas as pl
from jax.experimental.pallas import tpu as pltpu
from jax.experimental.pallas import tpu_sc as plsc

N, L, D1, D2 = 20000, 1024, 512, 256
K_SEL = 8
BLK = 2000
NBLK = N // BLK

_NEG_BIG = float(-3e38)
_POS_BIG = float(3e38)

SC_NW = 16
SC_ROWS_TOT = 1280
SC_ROWS = SC_ROWS_TOT // SC_NW


def _pass1_kernel(h_ref, w1_ref, b1_ref, wa_ref, ba_ref, wb_ref, bb_ref,
                  wc_ref, bc_ref, scores_ref, m_out_ref,
                  macc_ref, mmax_ref, denom_ref):
    i = pl.program_id(0)

    @pl.when(i == 0)
    def _init():
        macc_ref[...] = jnp.zeros_like(macc_ref)
        mmax_ref[...] = jnp.full((1, 1), -1e30, jnp.float32)
        denom_ref[...] = jnp.zeros((1, 1), jnp.float32)

    h1 = jax.nn.relu(
        lax.dot_general(h_ref[...].astype(jnp.bfloat16),
                        w1_ref[...].astype(jnp.bfloat16),
                        (((1,), (0,)), ((), ())),
                        preferred_element_type=jnp.float32)
        + b1_ref[...])
    h1b = h1.astype(jnp.bfloat16)
    a = jnp.tanh(
        lax.dot_general(h1b, wa_ref[...].astype(jnp.bfloat16),
                        (((1,), (0,)), ((), ())),
                        preferred_element_type=jnp.float32) + ba_ref[...])
    g = jax.nn.sigmoid(
        lax.dot_general(h1b, wb_ref[...].astype(jnp.bfloat16),
                        (((1,), (0,)), ((), ())),
                        preferred_element_type=jnp.float32) + bb_ref[...])
    ag = (a * g).astype(jnp.bfloat16)
    s = (lax.dot_general(ag, wc_ref[...].astype(jnp.bfloat16),
                         (((1,), (0,)), ((), ())),
                         preferred_element_type=jnp.float32)
         + bc_ref[0, 0])
    scores_ref[...] = s

    bmax = jnp.max(s, axis=0, keepdims=True)
    old_max = mmax_ref[...]
    new_max = jnp.maximum(old_max, bmax)
    alpha = jnp.exp(old_max - new_max)
    p = jnp.exp(s - new_max)
    denom_ref[...] = denom_ref[...] * alpha + jnp.sum(p, axis=0, keepdims=True)
    pw = jnp.sum(p * h1, axis=0, keepdims=True)
    macc_ref[...] = macc_ref[...] * alpha + pw
    mmax_ref[...] = new_max

    @pl.when(i == NBLK - 1)
    def _finish():
        m_out_ref[...] = macc_ref[...] / denom_ref[...]


def _sc_body(scores_ref, h_ref, rows_ref, tk_ref, ti_ref, bk_ref, bi_ref,
             chunk_ref, pub_ref, g1_ref, g2_ref, sem):
    w = lax.axis_index("s")
    base = w * SC_ROWS
    lane = lax.iota(jnp.int32, 16)

    pltpu.sync_copy(scores_ref.at[pl.ds(base * 16, SC_ROWS * 16)], chunk_ref)

    neg = jnp.full((16,), _NEG_BIG, jnp.float32)
    zid = jnp.zeros((16,), jnp.int32)

    def _scan(j, carry):
        kt, it, kb, ib = carry
        x = chunk_ref[pl.ds(j * 16, 16)]
        pid = (base + j) * 16 + lane
        valid = pid < N
        xt = jnp.where(valid, x, _NEG_BIG)
        xb = jnp.where(valid, -x, _NEG_BIG)
        xs, ps = plsc.sort_key_val(xt, pid)
        m = kt >= xs
        kt, it = plsc.sort_key_val(jnp.where(m, kt, xs),
                                   jnp.where(m, it, ps), descending=True)
        xs2, ps2 = plsc.sort_key_val(xb, pid)
        m2 = kb >= xs2
        kb, ib = plsc.sort_key_val(jnp.where(m2, kb, xs2),
                                   jnp.where(m2, ib, ps2), descending=True)
        return (kt, it, kb, ib)

    kt, it, kb, ib = lax.fori_loop(0, SC_ROWS, _scan, (neg, zid, neg, zid))

    for ref, vec in ((tk_ref, kt), (ti_ref, it.astype(jnp.float32)),
                     (bk_ref, kb), (bi_ref, ib.astype(jnp.float32))):
        pub_ref[...] = vec
        pltpu.sync_copy(pub_ref, ref.at[pl.ds(w * 16, 16)])
    plsc.subcore_barrier()

    @pl.when(w == 0)
    def _merge():
        def _merge2(ka, ia, kb2, ib2):
            kr = lax.rev(kb2, (0,))
            ir = lax.rev(ib2, (0,))
            m = ka >= kr
            return plsc.sort_key_val(jnp.where(m, ka, kr),
                                     jnp.where(m, ia, ir), descending=True)

        def _tree(kref, iref):
            ks, is_ = [], []
            for t in range(SC_NW):
                pltpu.sync_copy(kref.at[pl.ds(t * 16, 16)], pub_ref)
                ks.append(pub_ref[...])
                pltpu.sync_copy(iref.at[pl.ds(t * 16, 16)], pub_ref)
                is_.append(pub_ref[...].astype(jnp.int32))
            while len(ks) > 1:
                nk, ni = [], []
                for t in range(0, len(ks), 2):
                    a, b = _merge2(ks[t], is_[t], ks[t + 1], is_[t + 1])
                    nk.append(a)
                    ni.append(b)
                ks, is_ = nk, ni
            return ks[0], is_[0]

        _, itop = _tree(tk_ref, ti_ref)
        _, ibot = _tree(bk_ref, bi_ref)
        itop = jnp.where(lane < K_SEL, itop, 0)
        ibot = jnp.where(lane < K_SEL, ibot, 0)
        pltpu.async_copy(h_ref.at[itop], g1_ref, sem).wait()
        pltpu.async_copy(h_ref.at[ibot], g2_ref, sem).wait()
        pltpu.sync_copy(g1_ref.at[pl.ds(0, K_SEL)], rows_ref.at[pl.ds(0, K_SEL)])
        pltpu.sync_copy(g2_ref.at[pl.ds(0, K_SEL)],
                        rows_ref.at[pl.ds(K_SEL, K_SEL)])


def _classify_kernel(label_ref, rows_ref, w1_ref, b1_ref, wi0_ref, bi0_ref,
                     wi1_ref, bi1_ref, inst_ref):
    hsel = rows_ref[...].astype(jnp.bfloat16)
    h1s = jax.nn.relu(
        lax.dot_general(hsel, w1_ref[...].astype(jnp.bfloat16),
                        (((1,), (0,)), ((), ())),
                        preferred_element_type=jnp.float32) + b1_ref[...])
    h1sb = h1s.astype(jnp.bfloat16)
    logits0 = lax.dot_general(h1sb, wi0_ref[...].astype(jnp.bfloat16),
                              (((1,), (0,)), ((), ())),
                              preferred_element_type=jnp.float32) + bi0_ref[...]
    logits1 = lax.dot_general(h1sb, wi1_ref[...].astype(jnp.bfloat16),
                              (((1,), (0,)), ((), ())),
                              preferred_element_type=jnp.float32) + bi1_ref[...]
    inst_ref[...] = jnp.where(label_ref[0] == 0, logits0, logits1)


@functools.partial(jax.jit, static_argnames=())
def kernel(h, label, W1, b1, Wa, ba, Wb, bb, Wc, bc, Wi0, bi0, Wi1, bi1):
    b1r = b1.reshape(1, D1)
    bar = ba.reshape(1, D2)
    bbr = bb.reshape(1, D2)
    bcr = bc.reshape(1, 1)
    bi0r = bi0.reshape(1, 2)
    bi1r = bi1.reshape(1, 2)

    scores, M = pl.pallas_call(
        _pass1_kernel,
        grid=(NBLK,),
        in_specs=[
            pl.BlockSpec((BLK, L), lambda i: (i, 0)),
            pl.BlockSpec((L, D1), lambda i: (0, 0)),
            pl.BlockSpec((1, D1), lambda i: (0, 0)),
            pl.BlockSpec((D1, D2), lambda i: (0, 0)),
            pl.BlockSpec((1, D2), lambda i: (0, 0)),
            pl.BlockSpec((D1, D2), lambda i: (0, 0)),
            pl.BlockSpec((1, D2), lambda i: (0, 0)),
            pl.BlockSpec((D2, 1), lambda i: (0, 0)),
            pl.BlockSpec((1, 1), lambda i: (0, 0)),
        ],
        out_specs=[
            pl.BlockSpec((BLK, 1), lambda i: (i, 0)),
            pl.BlockSpec((1, D1), lambda i: (0, 0)),
        ],
        out_shape=[
            jax.ShapeDtypeStruct((N, 1), jnp.float32),
            jax.ShapeDtypeStruct((1, D1), jnp.float32),
        ],
        scratch_shapes=[
            pltpu.VMEM((1, D1), jnp.float32),
            pltpu.VMEM((1, 1), jnp.float32),
            pltpu.VMEM((1, 1), jnp.float32),
        ],
    )(h, W1, b1r, Wa, bar, Wb, bbr, Wc, bcr)

    scores_pad = jnp.concatenate(
        [scores.reshape(-1), jnp.zeros((SC_ROWS_TOT * 16 - N,), jnp.float32)])

    mesh = plsc.VectorSubcoreMesh(core_axis_name="c", subcore_axis_name="s",
                                  num_cores=1)
    sc_call = pl.kernel(
        _sc_body,
        out_type=[
            jax.ShapeDtypeStruct((2 * K_SEL, L), jnp.float32),
            jax.ShapeDtypeStruct((SC_NW * 16,), jnp.float32),
            jax.ShapeDtypeStruct((SC_NW * 16,), jnp.float32),
            jax.ShapeDtypeStruct((SC_NW * 16,), jnp.float32),
            jax.ShapeDtypeStruct((SC_NW * 16,), jnp.float32),
        ],
        mesh=mesh,
        scratch_types=[
            pltpu.VMEM((SC_ROWS * 16,), jnp.float32),
            pltpu.VMEM((16,), jnp.float32),
            pltpu.VMEM((2 * K_SEL, L), jnp.float32),
            pltpu.VMEM((2 * K_SEL, L), jnp.float32),
            pltpu.SemaphoreType.DMA,
        ],
        compiler_params=pltpu.CompilerParams(needs_layout_passes=False),
    )
    rows, _, _, _, _ = sc_call(scores_pad, h)

    inst = pl.pallas_call(
        _classify_kernel,
        in_specs=[
            pl.BlockSpec(memory_space=pltpu.MemorySpace.SMEM),
            pl.BlockSpec((2 * K_SEL, L), lambda: (0, 0)),
            pl.BlockSpec((L, D1), lambda: (0, 0)),
            pl.BlockSpec((1, D1), lambda: (0, 0)),
            pl.BlockSpec((D1, 2), lambda: (0, 0)),
            pl.BlockSpec((1, 2), lambda: (0, 0)),
            pl.BlockSpec((D1, 2), lambda: (0, 0)),
            pl.BlockSpec((1, 2), lambda: (0, 0)),
        ],
        out_specs=pl.BlockSpec((2 * K_SEL, 2), lambda: (0, 0)),
        out_shape=jax.ShapeDtypeStruct((2 * K_SEL, 2), jnp.float32),
    )(label, rows, W1, b1r, Wi0, bi0r, Wi1, bi1r)

    return jnp.concatenate([M.reshape(-1), inst.reshape(-1)])


# speedup vs baseline: 1.8065x; 1.8065x over previous
import functools

import jax
import jax.numpy as jnp
from jax.experimental import pallas as pl
from jax.experimental.pallas import tpu as pltpu

N, L, D1, D2 = 20000, 1024, 512, 256
K_SEL = 8
BLK = 2000
NBLK = N // BLK

_NEG_BIG = float(-3e38)
_POS_BIG = float(3e38)


def _clam_kernel(label_ref, h_ref, w1_ref, b1_ref, wa_ref, ba_ref, wb_ref,
                 bb_ref, wc_ref, bc_ref, hfull_ref, wi0_ref, bi0_ref,
                 wi1_ref, bi1_ref, m_out_ref, inst_ref,
                 scores_ref, macc_ref, mmax_ref, denom_ref, rows_ref, sems):
    i = pl.program_id(0)

    @pl.when(i == 0)
    def _init():
        macc_ref[...] = jnp.zeros_like(macc_ref)
        mmax_ref[...] = jnp.full((1, 1), -1e30, jnp.float32)
        denom_ref[...] = jnp.zeros((1, 1), jnp.float32)

    h1 = jax.nn.relu(
        jax.lax.dot_general(h_ref[...].astype(jnp.bfloat16),
                            w1_ref[...].astype(jnp.bfloat16),
                            (((1,), (0,)), ((), ())),
                            preferred_element_type=jnp.float32)
        + b1_ref[...])
    h1b = h1.astype(jnp.bfloat16)
    a = jnp.tanh(
        jax.lax.dot_general(h1b, wa_ref[...].astype(jnp.bfloat16), (((1,), (0,)), ((), ())),
                            preferred_element_type=jnp.float32) + ba_ref[...])
    g = jax.nn.sigmoid(
        jax.lax.dot_general(h1b, wb_ref[...].astype(jnp.bfloat16), (((1,), (0,)), ((), ())),
                            preferred_element_type=jnp.float32) + bb_ref[...])
    ag = (a * g).astype(jnp.bfloat16)
    s = (jax.lax.dot_general(ag, wc_ref[...].astype(jnp.bfloat16), (((1,), (0,)), ((), ())),
                             preferred_element_type=jnp.float32)
         + bc_ref[0, 0])
    col = jax.lax.broadcasted_iota(jnp.int32, (BLK, NBLK), 1)
    scores_ref[...] = jnp.where(col == i, s, scores_ref[...])

    bmax = jnp.max(s, axis=0, keepdims=True)
    old_max = mmax_ref[...]
    new_max = jnp.maximum(old_max, bmax)
    alpha = jnp.exp(old_max - new_max)
    p = jnp.exp(s - new_max)
    denom_ref[...] = denom_ref[...] * alpha + jnp.sum(p, axis=0, keepdims=True)
    pw = jnp.sum(p * h1, axis=0, keepdims=True)
    macc_ref[...] = macc_ref[...] * alpha + pw
    mmax_ref[...] = new_max

    @pl.when(i == NBLK - 1)
    def _finish():
        m_out_ref[...] = macc_ref[...] / denom_ref[...]

        S = jnp.transpose(scores_ref[...])
        ids = (jax.lax.broadcasted_iota(jnp.int32, S.shape, 0) * BLK
               + jax.lax.broadcasted_iota(jnp.int32, S.shape, 1))
        int_max = jnp.int32(2**31 - 1)

        St = S
        Sb = S
        for k in range(K_SEL):
            vt = jnp.max(St)
            vb = jnp.min(Sb)
            idxt = jnp.min(jnp.where(St == vt, ids, int_max))
            idxb = jnp.min(jnp.where(Sb == vb, ids, int_max))
            pltpu.make_async_copy(hfull_ref.at[pl.ds(idxt, 1), :],
                                  rows_ref.at[pl.ds(k, 1), :],
                                  sems.at[k]).start()
            pltpu.make_async_copy(hfull_ref.at[pl.ds(idxb, 1), :],
                                  rows_ref.at[pl.ds(K_SEL + k, 1), :],
                                  sems.at[K_SEL + k]).start()
            St = jnp.where(ids == idxt, _NEG_BIG, St)
            Sb = jnp.where(ids == idxb, _POS_BIG, Sb)

        for k in range(2 * K_SEL):
            pltpu.make_async_copy(hfull_ref.at[pl.ds(0, 1), :],
                                  rows_ref.at[pl.ds(k, 1), :],
                                  sems.at[k]).wait()

        hsel = rows_ref[...].astype(jnp.bfloat16)
        h1s = jax.nn.relu(
            jax.lax.dot_general(hsel, w1_ref[...].astype(jnp.bfloat16), (((1,), (0,)), ((), ())),
                                preferred_element_type=jnp.float32)
            + b1_ref[...])
        h1sb = h1s.astype(jnp.bfloat16)
        logits0 = jax.lax.dot_general(
            h1sb, wi0_ref[...].astype(jnp.bfloat16), (((1,), (0,)), ((), ())),
            preferred_element_type=jnp.float32) + bi0_ref[...]
        logits1 = jax.lax.dot_general(
            h1sb, wi1_ref[...].astype(jnp.bfloat16), (((1,), (0,)), ((), ())),
            preferred_element_type=jnp.float32) + bi1_ref[...]
        inst_ref[...] = jnp.where(label_ref[0] == 0, logits0, logits1)


@functools.partial(jax.jit, static_argnames=())
def kernel(h, label, W1, b1, Wa, ba, Wb, bb, Wc, bc, Wi0, bi0, Wi1, bi1):
    b1r = b1.reshape(1, D1)
    bar = ba.reshape(1, D2)
    bbr = bb.reshape(1, D2)
    bcr = bc.reshape(1, 1)
    bi0r = bi0.reshape(1, 2)
    bi1r = bi1.reshape(1, 2)

    M, inst = pl.pallas_call(
        _clam_kernel,
        grid=(NBLK,),
        in_specs=[
            pl.BlockSpec(memory_space=pltpu.MemorySpace.SMEM),
            pl.BlockSpec((BLK, L), lambda i: (i, 0)),
            pl.BlockSpec((L, D1), lambda i: (0, 0)),
            pl.BlockSpec((1, D1), lambda i: (0, 0)),
            pl.BlockSpec((D1, D2), lambda i: (0, 0)),
            pl.BlockSpec((1, D2), lambda i: (0, 0)),
            pl.BlockSpec((D1, D2), lambda i: (0, 0)),
            pl.BlockSpec((1, D2), lambda i: (0, 0)),
            pl.BlockSpec((D2, 1), lambda i: (0, 0)),
            pl.BlockSpec((1, 1), lambda i: (0, 0)),
            pl.BlockSpec(memory_space=pl.ANY),
            pl.BlockSpec((D1, 2), lambda i: (0, 0)),
            pl.BlockSpec((1, 2), lambda i: (0, 0)),
            pl.BlockSpec((D1, 2), lambda i: (0, 0)),
            pl.BlockSpec((1, 2), lambda i: (0, 0)),
        ],
        out_specs=[
            pl.BlockSpec((1, D1), lambda i: (0, 0)),
            pl.BlockSpec((2 * K_SEL, 2), lambda i: (0, 0)),
        ],
        out_shape=[
            jax.ShapeDtypeStruct((1, D1), jnp.float32),
            jax.ShapeDtypeStruct((2 * K_SEL, 2), jnp.float32),
        ],
        scratch_shapes=[
            pltpu.VMEM((BLK, NBLK), jnp.float32),
            pltpu.VMEM((1, D1), jnp.float32),
            pltpu.VMEM((1, 1), jnp.float32),
            pltpu.VMEM((1, 1), jnp.float32),
            pltpu.VMEM((2 * K_SEL, L), jnp.float32),
            pltpu.SemaphoreType.DMA((2 * K_SEL,)),
        ],
    )(label, h, W1, b1r, Wa, bar, Wb, bbr, Wc, bcr,
      h, Wi0, bi0r, Wi1, bi1r)

    return jnp.concatenate([M.reshape(-1), inst.reshape(-1)])
